# baseline (speedup 1.0000x reference)
import jax
import jax.numpy as jnp
from jax.experimental import pallas as pl
from jax.experimental.pallas import tpu as pltpu

N = 128
HID = 64


def _ggnn_kernel(J_ref, bs_ref, bcol_ref, W1t_ref, b1c_ref, W2t_ref,
                 b2c_ref, W3r_ref, b3s_ref, out_ref,
                 M0, M1, Mt0, Mt1, Old0, Old1, Oldt0, Oldt1):
    z = jnp.zeros((N, N), jnp.float32)
    M0[:] = z
    M1[:] = z
    Mt0[:] = z
    Mt1[:] = z

    W1t = W1t_ref[:]
    b1c = b1c_ref[:]
    W2t = W2t_ref[:]
    b2c = b2c_ref[:]
    W3r = W3r_ref[:]
    b3 = b3s_ref[0]

    def mlp_alpha(x_m, x_o, x_f, x_s):
        E = N * N
        Xt = jnp.concatenate([x_m.reshape(1, E), x_o.reshape(1, E),
                              x_f.reshape(1, E), x_s.reshape(1, E)],
                             axis=0)
        h = jnp.dot(W1t, Xt, preferred_element_type=jnp.float32) + b1c
        h = jnp.maximum(h, 0.0)
        h = jnp.dot(W2t, h, preferred_element_type=jnp.float32) + b2c
        h = jnp.maximum(h, 0.0)
        o = jnp.dot(W3r, h, preferred_element_type=jnp.float32) + b3
        return jax.nn.sigmoid(o).reshape(N, N)

    def outer(it, carry):
        Old0[:] = M0[:]
        Old1[:] = M1[:]
        Oldt0[:] = Mt0[:]
        Oldt1[:] = Mt1[:]

        lane = jax.lax.broadcasted_iota(jnp.int32, (1, N), 1)
        sub8 = jax.lax.broadcasted_iota(jnp.int32, (8, 1), 0)
        lane8 = jax.lax.broadcasted_iota(jnp.int32, (8, N), 1)
        for B in range(0, N, 8):
            slab0 = jnp.zeros((8, N), jnp.float32)
            slab1 = jnp.zeros((8, N), jnp.float32)
            pl_b = (lane8 - B == sub8).astype(jnp.float32)
            ge_b = lane >= B
            for r in range(8):
                i = B + r
                inc0 = Mt0[i:i + 1, :]
                inc1 = Mt1[i:i + 1, :]
                if r > 0:
                    is_i = lane8 == i
                    fresh = ge_b & (lane < i)
                    rs0 = jnp.sum(jnp.where(is_i, slab0, 0.0), axis=1,
                                  keepdims=True)
                    c0 = jnp.sum(rs0 * pl_b, axis=0, keepdims=True)
                    inc0 = jnp.where(fresh, c0, inc0)
                    rs1 = jnp.sum(jnp.where(is_i, slab1, 0.0), axis=1,
                                  keepdims=True)
                    c1 = jnp.sum(rs1 * pl_b, axis=0, keepdims=True)
                    inc1 = jnp.where(fresh, c1, inc1)
                bi = bs_ref[i]
                a0 = jnp.sum(inc0) - inc0 - bi
                a1 = jnp.sum(inc1) - inc1 + bi
                Jr = J_ref[i:i + 1, :]
                out0 = jnp.logaddexp(a0 + Jr, a1 - Jr)
                out1 = jnp.logaddexp(a0 - Jr, a1 + Jr)
                put = sub8 == r
                slab0 = jnp.where(put, out0, slab0)
                slab1 = jnp.where(put, out1, slab1)
            M0[B:B + 8, :] = slab0
            M1[B:B + 8, :] = slab1
            Mt0[:, B:B + 8] = slab0.T
            Mt1[:, B:B + 8] = slab1.T

        t0m = Mt0[:]
        t1m = Mt1[:]
        p0 = jnp.sum(t0m, axis=1, keepdims=True) - bcol_ref[:]
        m = M0[:]
        o = Old0[:]
        alpha = mlp_alpha(m, o, jnp.broadcast_to(p0, (N, N)), t0m + t1m)
        M0[:] = (1.0 - alpha) * m + alpha * o
        At = alpha.T
        Mt0[:] = (1.0 - At) * t0m + At * Oldt0[:]

        t0m = Mt0[:]
        p1 = jnp.sum(t1m, axis=1, keepdims=True) + bcol_ref[:]
        m = M1[:]
        o = Old1[:]
        alpha = mlp_alpha(m, o, jnp.broadcast_to(p1, (N, N)), t1m + t0m)
        M1[:] = (1.0 - alpha) * m + alpha * o
        At = alpha.T
        Mt1[:] = (1.0 - At) * t1m + At * Oldt1[:]
        return carry

    jax.lax.fori_loop(0, 10, outer, 0, unroll=False)

    probs0 = jnp.sum(Mt0[:], axis=1, keepdims=True) - bcol_ref[:]
    probs1 = jnp.sum(Mt1[:], axis=1, keepdims=True) + bcol_ref[:]
    mx = jnp.maximum(probs0, probs1)
    e0 = jnp.exp(probs0 - mx)
    e1 = jnp.exp(probs1 - mx)
    s = e0 + e1
    out_ref[:, 0:1] = e0 / s
    out_ref[:, 1:2] = e1 / s


def kernel(J, b, W1, b1, W2, b2, W3, b3):
    J = J.astype(jnp.float32)
    b = b.astype(jnp.float32)
    bcol = b.reshape(N, 1)
    W1t = W1.T.astype(jnp.float32)
    b1c = b1.reshape(HID, 1).astype(jnp.float32)
    W2t = W2.T.astype(jnp.float32)
    b2c = b2.reshape(HID, 1).astype(jnp.float32)
    W3r = W3.T.astype(jnp.float32)
    b3s = b3.reshape(1).astype(jnp.float32)

    vmem = pl.BlockSpec(memory_space=pltpu.VMEM)
    smem = pl.BlockSpec(memory_space=pltpu.SMEM)
    return pl.pallas_call(
        _ggnn_kernel,
        out_shape=jax.ShapeDtypeStruct((N, 2), jnp.float32),
        in_specs=[vmem, smem, vmem, vmem, vmem, vmem, vmem, vmem, smem],
        out_specs=vmem,
        scratch_shapes=[pltpu.VMEM((N, N), jnp.float32)] * 8,
    )(J, b, bcol, W1t, b1c, W2t, b2c, W3r, b3s)


# speedup vs baseline: 1.3863x; 1.3863x over previous
import jax
import jax.numpy as jnp
from jax.experimental import pallas as pl
from jax.experimental.pallas import tpu as pltpu

N = 128
HID = 64


def _ggnn_kernel(J_ref, bs_ref, bcol_ref, W1t_ref, b1c_ref, W2t_ref,
                 b2c_ref, W3r_ref, b3s_ref, out_ref,
                 M0, M1, Mt0, Mt1, Old0, Old1, Oldt0, Oldt1):
    z = jnp.zeros((N, N), jnp.float32)
    M0[:] = z
    M1[:] = z
    Mt0[:] = z
    Mt1[:] = z

    W1t = W1t_ref[:]
    b1c = b1c_ref[:]
    W2t = W2t_ref[:]
    b2c = b2c_ref[:]
    W3r = W3r_ref[:]
    b3 = b3s_ref[0]

    def mlp_alpha(x_m, x_o, x_f, x_s):
        E = N * N
        Xt = jnp.concatenate([x_m.reshape(1, E), x_o.reshape(1, E),
                              x_f.reshape(1, E), x_s.reshape(1, E)],
                             axis=0)
        h = jnp.dot(W1t, Xt, preferred_element_type=jnp.float32) + b1c
        h = jnp.maximum(h, 0.0)
        h = jnp.dot(W2t, h, preferred_element_type=jnp.float32) + b2c
        h = jnp.maximum(h, 0.0)
        o = jnp.dot(W3r, h, preferred_element_type=jnp.float32) + b3
        return jax.nn.sigmoid(o).reshape(N, N)

    def outer(it, carry):
        Old0[:] = M0[:]
        Old1[:] = M1[:]
        Oldt0[:] = Mt0[:]
        Oldt1[:] = Mt1[:]

        lane = jax.lax.broadcasted_iota(jnp.int32, (1, N), 1)
        sub8 = jax.lax.broadcasted_iota(jnp.int32, (8, 1), 0)
        ones_mat = jnp.full((N, N), 1.0, jnp.float32)
        for B in range(0, N, 8):
            stale0 = Mt0[B:B + 8, :]
            stale1 = Mt1[B:B + 8, :]
            slab0 = jnp.zeros((8, N), jnp.float32)
            slab1 = jnp.zeros((8, N), jnp.float32)
            c0 = jnp.zeros((1, N), jnp.float32)
            c1 = jnp.zeros((1, N), jnp.float32)
            for r in range(8):
                i = B + r
                row0 = stale0[r:r + 1, :]
                row1 = stale1[r:r + 1, :]
                if r > 0:
                    fresh = (lane >= B) & (lane < i)
                    row0 = jnp.where(fresh, c0, row0)
                    row1 = jnp.where(fresh, c1, row1)
                s0 = jnp.dot(row0, ones_mat,
                             preferred_element_type=jnp.float32)
                s1 = jnp.dot(row1, ones_mat,
                             preferred_element_type=jnp.float32)
                bi = bs_ref[i]
                a0 = s0 - row0 - bi
                a1 = s1 - row1 + bi
                Jr = J_ref[i:i + 1, :]
                out0 = jnp.logaddexp(a0 + Jr, a1 - Jr)
                out1 = jnp.logaddexp(a0 - Jr, a1 + Jr)
                oneh = lane == i
                c0 = jnp.where(oneh, out0, c0)
                c1 = jnp.where(oneh, out1, c1)
                put = sub8 == r
                slab0 = jnp.where(put, out0, slab0)
                slab1 = jnp.where(put, out1, slab1)
            M0[B:B + 8, :] = slab0
            M1[B:B + 8, :] = slab1
            Mt0[:, B:B + 8] = slab0.T
            Mt1[:, B:B + 8] = slab1.T

        t0m = Mt0[:]
        t1m = Mt1[:]
        p0 = jnp.sum(t0m, axis=1, keepdims=True) - bcol_ref[:]
        m = M0[:]
        o = Old0[:]
        alpha = mlp_alpha(m, o, jnp.broadcast_to(p0, (N, N)), t0m + t1m)
        M0[:] = (1.0 - alpha) * m + alpha * o
        At = alpha.T
        Mt0[:] = (1.0 - At) * t0m + At * Oldt0[:]

        t0m = Mt0[:]
        p1 = jnp.sum(t1m, axis=1, keepdims=True) + bcol_ref[:]
        m = M1[:]
        o = Old1[:]
        alpha = mlp_alpha(m, o, jnp.broadcast_to(p1, (N, N)), t1m + t0m)
        M1[:] = (1.0 - alpha) * m + alpha * o
        At = alpha.T
        Mt1[:] = (1.0 - At) * t1m + At * Oldt1[:]
        return carry

    jax.lax.fori_loop(0, 10, outer, 0, unroll=False)

    probs0 = jnp.sum(Mt0[:], axis=1, keepdims=True) - bcol_ref[:]
    probs1 = jnp.sum(Mt1[:], axis=1, keepdims=True) + bcol_ref[:]
    mx = jnp.maximum(probs0, probs1)
    e0 = jnp.exp(probs0 - mx)
    e1 = jnp.exp(probs1 - mx)
    s = e0 + e1
    out_ref[:, 0:1] = e0 / s
    out_ref[:, 1:2] = e1 / s


def kernel(J, b, W1, b1, W2, b2, W3, b3):
    J = J.astype(jnp.float32)
    b = b.astype(jnp.float32)
    bcol = b.reshape(N, 1)
    W1t = W1.T.astype(jnp.float32)
    b1c = b1.reshape(HID, 1).astype(jnp.float32)
    W2t = W2.T.astype(jnp.float32)
    b2c = b2.reshape(HID, 1).astype(jnp.float32)
    W3r = W3.T.astype(jnp.float32)
    b3s = b3.reshape(1).astype(jnp.float32)

    vmem = pl.BlockSpec(memory_space=pltpu.VMEM)
    smem = pl.BlockSpec(memory_space=pltpu.SMEM)
    return pl.pallas_call(
        _ggnn_kernel,
        out_shape=jax.ShapeDtypeStruct((N, 2), jnp.float32),
        in_specs=[vmem, smem, vmem, vmem, vmem, vmem, vmem, vmem, smem],
        out_specs=vmem,
        scratch_shapes=[pltpu.VMEM((N, N), jnp.float32)] * 8,
    )(J, b, bcol, W1t, b1c, W2t, b2c, W3r, b3s)


# speedup vs baseline: 1.7495x; 1.2620x over previous
import jax
import jax.numpy as jnp
from jax.experimental import pallas as pl
from jax.experimental.pallas import tpu as pltpu

N = 128
HID = 64
BS = 8
NB = N // BS


def _ggnn_kernel(J_ref, bs_ref, Jd_ref, bcol_ref, W1t_ref, b1c_ref,
                 W2t_ref, b2c_ref, W3r_ref, b3s_ref, out_ref,
                 M0, M1, Mt0, Mt1, Old0, Old1, Oldt0, Oldt1,
                 T0A, T0B, T1A, T1B, sem):
    z = jnp.zeros((N, N), jnp.float32)
    M0[:] = z
    M1[:] = z
    Mt0[:] = z
    Mt1[:] = z

    W1t = W1t_ref[:]
    b1c = b1c_ref[:]
    W2t = W2t_ref[:]
    b2c = b2c_ref[:]
    W3r = W3r_ref[:]
    b3 = b3s_ref[0]

    lane = jax.lax.broadcasted_iota(jnp.int32, (1, N), 1)
    sub8 = jax.lax.broadcasted_iota(jnp.int32, (BS, 1), 0)
    ones_row = jnp.full((1, N), 1.0, jnp.float32)

    def mlp_alpha(x_m, x_o, x_f, x_s):
        E = N * N
        Xt = jnp.concatenate([x_m.reshape(1, E), x_o.reshape(1, E),
                              x_f.reshape(1, E), x_s.reshape(1, E)],
                             axis=0)
        h = jnp.dot(W1t, Xt, preferred_element_type=jnp.float32) + b1c
        h = jnp.maximum(h, 0.0)
        h = jnp.dot(W2t, h, preferred_element_type=jnp.float32) + b2c
        h = jnp.maximum(h, 0.0)
        o = jnp.dot(W3r, h, preferred_element_type=jnp.float32) + b3
        return jax.nn.sigmoid(o).reshape(N, N)

    def outer(it, carry):
        Old0[:] = M0[:]
        Old1[:] = M1[:]
        Oldt0[:] = Mt0[:]
        Oldt1[:] = Mt1[:]

        def row_copies(k):
            s = k * BS
            t0, t1 = (T0A, T1A) if k % 2 == 0 else (T0B, T1B)
            return (pltpu.make_async_copy(Mt0.at[pl.ds(s, BS), :], t0,
                                          sem),
                    pltpu.make_async_copy(Mt1.at[pl.ds(s, BS), :], t1,
                                          sem))

        pending = row_copies(0)
        for c in pending:
            c.start()

        RS0 = jnp.dot(ones_row, M0[:], preferred_element_type=jnp.float32)
        RS1 = jnp.dot(ones_row, M1[:], preferred_element_type=jnp.float32)

        for B in range(0, N, BS):
            k = B // BS
            for c in pending:
                c.wait()
            T0sm, T1sm = (T0A, T1A) if k % 2 == 0 else (T0B, T1B)
            if k + 1 < NB:
                pending = row_copies(k + 1)
                for c in pending:
                    c.start()
            stale0 = Mt0[B:B + BS, :]
            stale1 = Mt1[B:B + BS, :]
            oldM0 = M0[B:B + BS, :]
            oldM1 = M1[B:B + BS, :]
            slab0 = jnp.zeros((BS, N), jnp.float32)
            slab1 = jnp.zeros((BS, N), jnp.float32)
            g0 = {}
            g1 = {}
            for r in range(BS):
                i = B + r
                bi = bs_ref[i]
                s0v = jnp.broadcast_to(RS0[0:1, i:i + 1], (1, N))
                s1v = jnp.broadcast_to(RS1[0:1, i:i + 1], (1, N))
                if r > 0:
                    pT0 = T0sm[r, B + 0]
                    pT1 = T1sm[r, B + 0]
                    corr0 = g0[(0, r)]
                    corr1 = g1[(0, r)]
                    for q in range(1, r):
                        pT0 = pT0 + T0sm[r, B + q]
                        pT1 = pT1 + T1sm[r, B + q]
                        corr0 = corr0 + g0[(q, r)]
                        corr1 = corr1 + g1[(q, r)]
                    s0v = (s0v + corr0) - pT0
                    s1v = (s1v + corr1) - pT1
                row0 = stale0[r:r + 1, :]
                row1 = stale1[r:r + 1, :]
                for q in range(r):
                    lm = lane == B + q
                    row0 = jnp.where(lm, g0[(q, r)], row0)
                    row1 = jnp.where(lm, g1[(q, r)], row1)
                a0 = s0v - row0 - bi
                a1 = s1v - row1 + bi
                Jr = J_ref[i:i + 1, :]
                out0 = jnp.logaddexp(a0 + Jr, a1 - Jr)
                out1 = jnp.logaddexp(a0 - Jr, a1 + Jr)
                put = sub8 == r
                slab0 = jnp.where(put, out0, slab0)
                slab1 = jnp.where(put, out1, slab1)
                for rp in range(r + 1, BS):
                    T0s = T0sm[r, B + rp]
                    T1s = T1sm[r, B + rp]
                    Js = Jd_ref[i, rp]
                    a0g = s0v - T0s - bi
                    a1g = s1v - T1s + bi
                    g0[(r, rp)] = jnp.logaddexp(a0g + Js, a1g - Js)
                    g1[(r, rp)] = jnp.logaddexp(a0g - Js, a1g + Js)
            M0[B:B + BS, :] = slab0
            M1[B:B + BS, :] = slab1
            Mt0[:, B:B + BS] = slab0.T
            Mt1[:, B:B + BS] = slab1.T
            RS0 = RS0 + (jnp.sum(slab0, axis=0, keepdims=True)
                         - jnp.sum(oldM0, axis=0, keepdims=True))
            RS1 = RS1 + (jnp.sum(slab1, axis=0, keepdims=True)
                         - jnp.sum(oldM1, axis=0, keepdims=True))

        t0m = Mt0[:]
        t1m = Mt1[:]
        p0 = jnp.sum(t0m, axis=1, keepdims=True) - bcol_ref[:]
        m = M0[:]
        o = Old0[:]
        alpha = mlp_alpha(m, o, jnp.broadcast_to(p0, (N, N)), t0m + t1m)
        M0[:] = (1.0 - alpha) * m + alpha * o
        At = alpha.T
        Mt0[:] = (1.0 - At) * t0m + At * Oldt0[:]

        t0m = Mt0[:]
        p1 = jnp.sum(t1m, axis=1, keepdims=True) + bcol_ref[:]
        m = M1[:]
        o = Old1[:]
        alpha = mlp_alpha(m, o, jnp.broadcast_to(p1, (N, N)), t1m + t0m)
        M1[:] = (1.0 - alpha) * m + alpha * o
        At = alpha.T
        Mt1[:] = (1.0 - At) * t1m + At * Oldt1[:]
        return carry

    jax.lax.fori_loop(0, 10, outer, 0, unroll=False)

    probs0 = jnp.sum(Mt0[:], axis=1, keepdims=True) - bcol_ref[:]
    probs1 = jnp.sum(Mt1[:], axis=1, keepdims=True) + bcol_ref[:]
    mx = jnp.maximum(probs0, probs1)
    e0 = jnp.exp(probs0 - mx)
    e1 = jnp.exp(probs1 - mx)
    s = e0 + e1
    out_ref[:, 0:1] = e0 / s
    out_ref[:, 1:2] = e1 / s


def kernel(J, b, W1, b1, W2, b2, W3, b3):
    J = J.astype(jnp.float32)
    b = b.astype(jnp.float32)
    bcol = b.reshape(N, 1)
    Jd = J.reshape(NB, BS, NB, BS)[jnp.arange(NB), :, jnp.arange(NB), :]
    Jd = Jd.reshape(N, BS)
    W1t = W1.T.astype(jnp.float32)
    b1c = b1.reshape(HID, 1).astype(jnp.float32)
    W2t = W2.T.astype(jnp.float32)
    b2c = b2.reshape(HID, 1).astype(jnp.float32)
    W3r = W3.T.astype(jnp.float32)
    b3s = b3.reshape(1).astype(jnp.float32)

    vmem = pl.BlockSpec(memory_space=pltpu.VMEM)
    smem = pl.BlockSpec(memory_space=pltpu.SMEM)
    return pl.pallas_call(
        _ggnn_kernel,
        out_shape=jax.ShapeDtypeStruct((N, 2), jnp.float32),
        in_specs=[vmem, smem, smem, vmem, vmem, vmem, vmem, vmem, vmem,
                  smem],
        out_specs=vmem,
        scratch_shapes=[pltpu.VMEM((N, N), jnp.float32)] * 8
        + [pltpu.SMEM((BS, N), jnp.float32)] * 4
        + [pltpu.SemaphoreType.DMA],
    )(J, b, Jd, bcol, W1t, b1c, W2t, b2c, W3r, b3s)


# speedup vs baseline: 2.4495x; 1.4001x over previous
import jax
import jax.numpy as jnp
from jax.experimental import pallas as pl
from jax.experimental.pallas import tpu as pltpu

N = 128
HID = 64
BS = 8
NB = N // BS


def _ggnn_kernel(J_ref, bs_ref, Jd_ref, bcol_ref, W1t_ref, b1c_ref,
                 W2t_ref, b2c_ref, W3r_ref, b3s_ref, out_ref,
                 M0, M1, Mt0, Mt1, Old0, Old1, Oldt0, Oldt1,
                 T0A, T0B, T0C, T1A, T1B, T1C, sem):
    z = jnp.zeros((N, N), jnp.float32)
    M0[:] = z
    M1[:] = z
    Mt0[:] = z
    Mt1[:] = z

    W1t = W1t_ref[:]
    b1c = b1c_ref[:]
    W2t = W2t_ref[:]
    b2c = b2c_ref[:]
    W3r = W3r_ref[:]
    b3 = b3s_ref[0]

    lane = jax.lax.broadcasted_iota(jnp.int32, (1, N), 1)
    sub8 = jax.lax.broadcasted_iota(jnp.int32, (BS, 1), 0)
    ones_row = jnp.full((1, N), 1.0, jnp.float32)

    def mlp_alpha(x_m, x_o, x_f, x_s):
        E = N * N
        Xt = jnp.concatenate([x_m.reshape(1, E), x_o.reshape(1, E),
                              x_f.reshape(1, E), x_s.reshape(1, E)],
                             axis=0)
        h = jnp.dot(W1t, Xt, preferred_element_type=jnp.float32) + b1c
        h = jnp.maximum(h, 0.0)
        h = jnp.dot(W2t, h, preferred_element_type=jnp.float32) + b2c
        h = jnp.maximum(h, 0.0)
        o = jnp.dot(W3r, h, preferred_element_type=jnp.float32) + b3
        return jax.nn.sigmoid(o).reshape(N, N)

    def outer(it, carry):
        Old0[:] = M0[:]
        Old1[:] = M1[:]
        Oldt0[:] = Mt0[:]
        Oldt1[:] = Mt1[:]

        t0bufs = (T0A, T0B, T0C)
        t1bufs = (T1A, T1B, T1C)

        def row_copies(k):
            s = k * BS
            return (pltpu.make_async_copy(Mt0.at[pl.ds(s, BS), :],
                                          t0bufs[k % 3], sem),
                    pltpu.make_async_copy(Mt1.at[pl.ds(s, BS), :],
                                          t1bufs[k % 3], sem))

        pending = [row_copies(0), row_copies(1)]
        for pc in pending:
            for c in pc:
                c.start()

        RS0 = jnp.dot(ones_row, M0[:], preferred_element_type=jnp.float32)
        RS1 = jnp.dot(ones_row, M1[:], preferred_element_type=jnp.float32)

        for B in range(0, N, BS):
            k = B // BS
            for c in pending.pop(0):
                c.wait()
            T0sm, T1sm = t0bufs[k % 3], t1bufs[k % 3]
            if k + 2 < NB:
                nxt = row_copies(k + 2)
                pending.append(nxt)
                for c in nxt:
                    c.start()
            stale0 = Mt0[B:B + BS, :]
            stale1 = Mt1[B:B + BS, :]
            oldM0 = M0[B:B + BS, :]
            oldM1 = M1[B:B + BS, :]
            slab0 = jnp.zeros((BS, N), jnp.float32)
            slab1 = jnp.zeros((BS, N), jnp.float32)
            g0 = {}
            g1 = {}
            for r in range(BS):
                i = B + r
                bi = bs_ref[i]
                s0v = jnp.broadcast_to(RS0[0:1, i:i + 1], (1, N))
                s1v = jnp.broadcast_to(RS1[0:1, i:i + 1], (1, N))
                if r > 0:
                    pT0 = T0sm[r, B + 0]
                    pT1 = T1sm[r, B + 0]
                    corr0 = g0[(0, r)]
                    corr1 = g1[(0, r)]
                    for q in range(1, r):
                        pT0 = pT0 + T0sm[r, B + q]
                        pT1 = pT1 + T1sm[r, B + q]
                        corr0 = corr0 + g0[(q, r)]
                        corr1 = corr1 + g1[(q, r)]
                    s0v = (s0v + corr0) - pT0
                    s1v = (s1v + corr1) - pT1
                row0 = stale0[r:r + 1, :]
                row1 = stale1[r:r + 1, :]
                for q in range(r):
                    lm = lane == B + q
                    row0 = jnp.where(lm, g0[(q, r)], row0)
                    row1 = jnp.where(lm, g1[(q, r)], row1)
                a0 = s0v - row0 - bi
                a1 = s1v - row1 + bi
                Jr = J_ref[i:i + 1, :]
                out0 = jnp.logaddexp(a0 + Jr, a1 - Jr)
                out1 = jnp.logaddexp(a0 - Jr, a1 + Jr)
                put = sub8 == r
                slab0 = jnp.where(put, out0, slab0)
                slab1 = jnp.where(put, out1, slab1)
                for rp in range(r + 1, BS):
                    T0s = T0sm[r, B + rp]
                    T1s = T1sm[r, B + rp]
                    Js = Jd_ref[i, rp]
                    a0g = s0v - T0s - bi
                    a1g = s1v - T1s + bi
                    g0[(r, rp)] = jnp.logaddexp(a0g + Js, a1g - Js)
                    g1[(r, rp)] = jnp.logaddexp(a0g - Js, a1g + Js)
            M0[B:B + BS, :] = slab0
            M1[B:B + BS, :] = slab1
            Mt0[:, B:B + BS] = slab0.T
            Mt1[:, B:B + BS] = slab1.T
            RS0 = RS0 + (jnp.sum(slab0, axis=0, keepdims=True)
                         - jnp.sum(oldM0, axis=0, keepdims=True))
            RS1 = RS1 + (jnp.sum(slab1, axis=0, keepdims=True)
                         - jnp.sum(oldM1, axis=0, keepdims=True))

        t0m = Mt0[:]
        t1m = Mt1[:]
        p0 = jnp.sum(t0m, axis=1, keepdims=True) - bcol_ref[:]
        m = M0[:]
        o = Old0[:]
        alpha = mlp_alpha(m, o, jnp.broadcast_to(p0, (N, N)), t0m + t1m)
        M0[:] = (1.0 - alpha) * m + alpha * o
        At = alpha.T
        Mt0[:] = (1.0 - At) * t0m + At * Oldt0[:]

        t0m = Mt0[:]
        p1 = jnp.sum(t1m, axis=1, keepdims=True) + bcol_ref[:]
        m = M1[:]
        o = Old1[:]
        alpha = mlp_alpha(m, o, jnp.broadcast_to(p1, (N, N)), t1m + t0m)
        M1[:] = (1.0 - alpha) * m + alpha * o
        At = alpha.T
        Mt1[:] = (1.0 - At) * t1m + At * Oldt1[:]
        return carry

    jax.lax.fori_loop(0, 10, outer, 0, unroll=False)

    probs0 = jnp.sum(Mt0[:], axis=1, keepdims=True) - bcol_ref[:]
    probs1 = jnp.sum(Mt1[:], axis=1, keepdims=True) + bcol_ref[:]
    mx = jnp.maximum(probs0, probs1)
    e0 = jnp.exp(probs0 - mx)
    e1 = jnp.exp(probs1 - mx)
    s = e0 + e1
    out_ref[:, 0:1] = e0 / s
    out_ref[:, 1:2] = e1 / s


def kernel(J, b, W1, b1, W2, b2, W3, b3):
    J = J.astype(jnp.float32)
    b = b.astype(jnp.float32)
    bcol = b.reshape(N, 1)
    Jd = J.reshape(NB, BS, NB, BS)[jnp.arange(NB), :, jnp.arange(NB), :]
    Jd = Jd.reshape(N, BS)
    W1t = W1.T.astype(jnp.float32)
    b1c = b1.reshape(HID, 1).astype(jnp.float32)
    W2t = W2.T.astype(jnp.float32)
    b2c = b2.reshape(HID, 1).astype(jnp.float32)
    W3r = W3.T.astype(jnp.float32)
    b3s = b3.reshape(1).astype(jnp.float32)

    vmem = pl.BlockSpec(memory_space=pltpu.VMEM)
    smem = pl.BlockSpec(memory_space=pltpu.SMEM)
    return pl.pallas_call(
        _ggnn_kernel,
        out_shape=jax.ShapeDtypeStruct((N, 2), jnp.float32),
        in_specs=[vmem, smem, smem, vmem, vmem, vmem, vmem, vmem, vmem,
                  smem],
        out_specs=vmem,
        scratch_shapes=[pltpu.VMEM((N, N), jnp.float32)] * 8
        + [pltpu.SMEM((BS, N), jnp.float32)] * 6
        + [pltpu.SemaphoreType.DMA],
    )(J, b, Jd, bcol, W1t, b1c, W2t, b2c, W3r, b3s)


# speedup vs baseline: 2.4513x; 1.0008x over previous
import jax
import jax.numpy as jnp
from jax.experimental import pallas as pl
from jax.experimental.pallas import tpu as pltpu

N = 128
HID = 64
BS = 8
NB = N // BS


def _ggnn_kernel(J_ref, bs_ref, Jd_ref, bcol_ref, W1t_ref, b1c_ref,
                 W2t_ref, b2c_ref, W3r_ref, b3s_ref, out_ref,
                 M0, M1, Mt0, Mt1, Old0, Old1,
                 T0A, T0B, T0C, T1A, T1B, T1C, sem):
    z = jnp.zeros((N, N), jnp.float32)
    M0[:] = z
    M1[:] = z
    Mt0[:] = z
    Mt1[:] = z

    W1t = W1t_ref[:]
    b1c = b1c_ref[:]
    W2t = W2t_ref[:]
    b2c = b2c_ref[:]
    W3r = W3r_ref[:]
    b3 = b3s_ref[0]

    lane = jax.lax.broadcasted_iota(jnp.int32, (1, N), 1)
    sub8 = jax.lax.broadcasted_iota(jnp.int32, (BS, 1), 0)
    ones_row = jnp.full((1, N), 1.0, jnp.float32)

    def mlp_alpha(x_m, x_o, x_f, x_s):
        E = N * N
        Xt = jnp.concatenate([x_m.reshape(1, E), x_o.reshape(1, E),
                              x_f.reshape(1, E), x_s.reshape(1, E)],
                             axis=0)
        h = jnp.dot(W1t, Xt, preferred_element_type=jnp.float32) + b1c
        h = jnp.maximum(h, 0.0)
        h = jnp.dot(W2t, h, preferred_element_type=jnp.float32) + b2c
        h = jnp.maximum(h, 0.0)
        o = jnp.dot(W3r, h, preferred_element_type=jnp.float32) + b3
        return jax.nn.sigmoid(o).reshape(N, N)

    def outer(it, carry):
        Old0[:] = M0[:]
        Old1[:] = M1[:]

        t0bufs = (T0A, T0B, T0C)
        t1bufs = (T1A, T1B, T1C)

        def row_copies(k):
            s = k * BS
            return (pltpu.make_async_copy(Mt0.at[pl.ds(s, BS), :],
                                          t0bufs[k % 3], sem),
                    pltpu.make_async_copy(Mt1.at[pl.ds(s, BS), :],
                                          t1bufs[k % 3], sem))

        pending = [row_copies(0), row_copies(1)]
        for pc in pending:
            for c in pc:
                c.start()

        RS0 = jnp.dot(ones_row, M0[:], preferred_element_type=jnp.float32)
        RS1 = jnp.dot(ones_row, M1[:], preferred_element_type=jnp.float32)

        for B in range(0, N, BS):
            k = B // BS
            for c in pending.pop(0):
                c.wait()
            T0sm, T1sm = t0bufs[k % 3], t1bufs[k % 3]
            if k + 2 < NB:
                nxt = row_copies(k + 2)
                pending.append(nxt)
                for c in nxt:
                    c.start()
            stale0 = Mt0[B:B + BS, :]
            stale1 = Mt1[B:B + BS, :]
            oldM0 = M0[B:B + BS, :]
            oldM1 = M1[B:B + BS, :]
            slab0 = jnp.zeros((BS, N), jnp.float32)
            slab1 = jnp.zeros((BS, N), jnp.float32)
            g0 = {}
            g1 = {}
            for r in range(BS):
                i = B + r
                bi = bs_ref[i]
                s0v = jnp.broadcast_to(RS0[0:1, i:i + 1], (1, N))
                s1v = jnp.broadcast_to(RS1[0:1, i:i + 1], (1, N))
                if r > 0:
                    pT0 = T0sm[r, B + 0]
                    pT1 = T1sm[r, B + 0]
                    corr0 = g0[(0, r)]
                    corr1 = g1[(0, r)]
                    for q in range(1, r):
                        pT0 = pT0 + T0sm[r, B + q]
                        pT1 = pT1 + T1sm[r, B + q]
                        corr0 = corr0 + g0[(q, r)]
                        corr1 = corr1 + g1[(q, r)]
                    s0v = (s0v + corr0) - pT0
                    s1v = (s1v + corr1) - pT1
                row0 = stale0[r:r + 1, :]
                row1 = stale1[r:r + 1, :]
                for q in range(r):
                    lm = lane == B + q
                    row0 = jnp.where(lm, g0[(q, r)], row0)
                    row1 = jnp.where(lm, g1[(q, r)], row1)
                a0 = s0v - row0 - bi
                a1 = s1v - row1 + bi
                Jr = J_ref[i:i + 1, :]
                out0 = jnp.logaddexp(a0 + Jr, a1 - Jr)
                out1 = jnp.logaddexp(a0 - Jr, a1 + Jr)
                put = sub8 == r
                slab0 = jnp.where(put, out0, slab0)
                slab1 = jnp.where(put, out1, slab1)
                for rp in range(r + 1, BS):
                    T0s = T0sm[r, B + rp]
                    T1s = T1sm[r, B + rp]
                    Js = Jd_ref[i, rp]
                    a0g = s0v - T0s - bi
                    a1g = s1v - T1s + bi
                    g0[(r, rp)] = jnp.logaddexp(a0g + Js, a1g - Js)
                    g1[(r, rp)] = jnp.logaddexp(a0g - Js, a1g + Js)
            M0[B:B + BS, :] = slab0
            M1[B:B + BS, :] = slab1
            Mt0[:, B:B + BS] = slab0.T
            Mt1[:, B:B + BS] = slab1.T
            RS0 = RS0 + (jnp.sum(slab0, axis=0, keepdims=True)
                         - jnp.sum(oldM0, axis=0, keepdims=True))
            RS1 = RS1 + (jnp.sum(slab1, axis=0, keepdims=True)
                         - jnp.sum(oldM1, axis=0, keepdims=True))

        t0m = Mt0[:]
        t1m = Mt1[:]
        p0 = jnp.sum(t0m, axis=1, keepdims=True) - bcol_ref[:]
        m = M0[:]
        o = Old0[:]
        alpha = mlp_alpha(m, o, jnp.broadcast_to(p0, (N, N)), t0m + t1m)
        new0 = (1.0 - alpha) * m + alpha * o
        M0[:] = new0
        Mt0[:] = new0.T

        t0m = Mt0[:]
        p1 = jnp.sum(t1m, axis=1, keepdims=True) + bcol_ref[:]
        m = M1[:]
        o = Old1[:]
        alpha = mlp_alpha(m, o, jnp.broadcast_to(p1, (N, N)), t1m + t0m)
        new1 = (1.0 - alpha) * m + alpha * o
        M1[:] = new1
        Mt1[:] = new1.T
        return carry

    jax.lax.fori_loop(0, 10, outer, 0, unroll=False)

    probs0 = jnp.sum(Mt0[:], axis=1, keepdims=True) - bcol_ref[:]
    probs1 = jnp.sum(Mt1[:], axis=1, keepdims=True) + bcol_ref[:]
    mx = jnp.maximum(probs0, probs1)
    e0 = jnp.exp(probs0 - mx)
    e1 = jnp.exp(probs1 - mx)
    s = e0 + e1
    out_ref[:, 0:1] = e0 / s
    out_ref[:, 1:2] = e1 / s


def kernel(J, b, W1, b1, W2, b2, W3, b3):
    J = J.astype(jnp.float32)
    b = b.astype(jnp.float32)
    bcol = b.reshape(N, 1)
    Jd = J.reshape(NB, BS, NB, BS)[jnp.arange(NB), :, jnp.arange(NB), :]
    Jd = Jd.reshape(N, BS)
    W1t = W1.T.astype(jnp.float32)
    b1c = b1.reshape(HID, 1).astype(jnp.float32)
    W2t = W2.T.astype(jnp.float32)
    b2c = b2.reshape(HID, 1).astype(jnp.float32)
    W3r = W3.T.astype(jnp.float32)
    b3s = b3.reshape(1).astype(jnp.float32)

    vmem = pl.BlockSpec(memory_space=pltpu.VMEM)
    smem = pl.BlockSpec(memory_space=pltpu.SMEM)
    return pl.pallas_call(
        _ggnn_kernel,
        out_shape=jax.ShapeDtypeStruct((N, 2), jnp.float32),
        in_specs=[vmem, smem, smem, vmem, vmem, vmem, vmem, vmem, vmem,
                  smem],
        out_specs=vmem,
        scratch_shapes=[pltpu.VMEM((N, N), jnp.float32)] * 6
        + [pltpu.SMEM((BS, N), jnp.float32)] * 6
        + [pltpu.SemaphoreType.DMA],
    )(J, b, Jd, bcol, W1t, b1c, W2t, b2c, W3r, b3s)


# speedup vs baseline: 2.4615x; 1.0042x over previous
import jax
import jax.numpy as jnp
from jax.experimental import pallas as pl
from jax.experimental.pallas import tpu as pltpu

N = 128
HID = 64
BS = 8
NB = N // BS


def _ggnn_kernel(J_ref, bs_ref, Jd_ref, bcol_ref, W1t_ref, b1c_ref,
                 W2t_ref, b2c_ref, W3r_ref, b3s_ref, out_ref,
                 M0, M1, Mt0, Mt1, Old0, Old1,
                 SGA, SGB, SGC, TA, TB, TC, sem):
    z = jnp.zeros((N, N), jnp.float32)
    M0[:] = z
    M1[:] = z
    Mt0[:] = z
    Mt1[:] = z

    W1t = W1t_ref[:]
    b1c = b1c_ref[:]
    W2t = W2t_ref[:]
    b2c = b2c_ref[:]
    W3r = W3r_ref[:]
    b3 = b3s_ref[0]

    lane = jax.lax.broadcasted_iota(jnp.int32, (1, N), 1)
    sub8 = jax.lax.broadcasted_iota(jnp.int32, (BS, 1), 0)
    ones_row = jnp.full((1, N), 1.0, jnp.float32)

    def mlp_alpha(x_m, x_o, x_f, x_s):
        E = N * N
        Xt = jnp.concatenate([x_m.reshape(1, E), x_o.reshape(1, E),
                              x_f.reshape(1, E), x_s.reshape(1, E)],
                             axis=0)
        h = jnp.dot(W1t, Xt, preferred_element_type=jnp.float32) + b1c
        h = jnp.maximum(h, 0.0)
        h = jnp.dot(W2t, h, preferred_element_type=jnp.float32) + b2c
        h = jnp.maximum(h, 0.0)
        o = jnp.dot(W3r, h, preferred_element_type=jnp.float32) + b3
        return jax.nn.sigmoid(o).reshape(N, N)

    def outer(it, carry):
        Old0[:] = M0[:]
        Old1[:] = M1[:]

        sgbufs = (SGA, SGB, SGC)
        tbufs = (TA, TB, TC)

        def row_copies(k):
            s = k * BS
            sg = sgbufs[k % 3]
            sg[0:BS, :] = Mt0[s:s + BS, :]
            sg[BS:2 * BS, :] = Mt1[s:s + BS, :]
            return pltpu.make_async_copy(sg, tbufs[k % 3], sem)

        pending = [row_copies(0), row_copies(1)]
        for c in pending:
            c.start()

        RS0 = jnp.dot(ones_row, M0[:], preferred_element_type=jnp.float32)
        RS1 = jnp.dot(ones_row, M1[:], preferred_element_type=jnp.float32)

        for B in range(0, N, BS):
            k = B // BS
            pending.pop(0).wait()
            Tsm = tbufs[k % 3]
            if k + 2 < NB:
                nxt = row_copies(k + 2)
                pending.append(nxt)
                nxt.start()
            stale0 = Mt0[B:B + BS, :]
            stale1 = Mt1[B:B + BS, :]
            oldM0 = M0[B:B + BS, :]
            oldM1 = M1[B:B + BS, :]
            slab0 = jnp.zeros((BS, N), jnp.float32)
            slab1 = jnp.zeros((BS, N), jnp.float32)
            g0 = {}
            g1 = {}
            for r in range(BS):
                i = B + r
                bi = bs_ref[i]
                s0v = jnp.broadcast_to(RS0[0:1, i:i + 1], (1, N))
                s1v = jnp.broadcast_to(RS1[0:1, i:i + 1], (1, N))
                if r > 0:
                    pT0 = Tsm[r, B + 0]
                    pT1 = Tsm[BS + r, B + 0]
                    corr0 = g0[(0, r)]
                    corr1 = g1[(0, r)]
                    for q in range(1, r):
                        pT0 = pT0 + Tsm[r, B + q]
                        pT1 = pT1 + Tsm[BS + r, B + q]
                        corr0 = corr0 + g0[(q, r)]
                        corr1 = corr1 + g1[(q, r)]
                    s0v = (s0v + corr0) - pT0
                    s1v = (s1v + corr1) - pT1
                row0 = stale0[r:r + 1, :]
                row1 = stale1[r:r + 1, :]
                for q in range(r):
                    lm = lane == B + q
                    row0 = jnp.where(lm, g0[(q, r)], row0)
                    row1 = jnp.where(lm, g1[(q, r)], row1)
                a0 = s0v - row0 - bi
                a1 = s1v - row1 + bi
                Jr = J_ref[i:i + 1, :]
                out0 = jnp.logaddexp(a0 + Jr, a1 - Jr)
                out1 = jnp.logaddexp(a0 - Jr, a1 + Jr)
                put = sub8 == r
                slab0 = jnp.where(put, out0, slab0)
                slab1 = jnp.where(put, out1, slab1)
                for rp in range(r + 1, BS):
                    T0s = Tsm[r, B + rp]
                    T1s = Tsm[BS + r, B + rp]
                    Js = Jd_ref[i, rp]
                    a0g = s0v - T0s - bi
                    a1g = s1v - T1s + bi
                    g0[(r, rp)] = jnp.logaddexp(a0g + Js, a1g - Js)
                    g1[(r, rp)] = jnp.logaddexp(a0g - Js, a1g + Js)
            M0[B:B + BS, :] = slab0
            M1[B:B + BS, :] = slab1
            Mt0[:, B:B + BS] = slab0.T
            Mt1[:, B:B + BS] = slab1.T
            RS0 = RS0 + (jnp.sum(slab0, axis=0, keepdims=True)
                         - jnp.sum(oldM0, axis=0, keepdims=True))
            RS1 = RS1 + (jnp.sum(slab1, axis=0, keepdims=True)
                         - jnp.sum(oldM1, axis=0, keepdims=True))

        t0m = Mt0[:]
        t1m = Mt1[:]
        p0 = jnp.sum(t0m, axis=1, keepdims=True) - bcol_ref[:]
        m = M0[:]
        o = Old0[:]
        alpha = mlp_alpha(m, o, jnp.broadcast_to(p0, (N, N)), t0m + t1m)
        new0 = (1.0 - alpha) * m + alpha * o
        M0[:] = new0
        Mt0[:] = new0.T

        t0m = Mt0[:]
        p1 = jnp.sum(t1m, axis=1, keepdims=True) + bcol_ref[:]
        m = M1[:]
        o = Old1[:]
        alpha = mlp_alpha(m, o, jnp.broadcast_to(p1, (N, N)), t1m + t0m)
        new1 = (1.0 - alpha) * m + alpha * o
        M1[:] = new1
        Mt1[:] = new1.T
        return carry

    jax.lax.fori_loop(0, 10, outer, 0, unroll=False)

    probs0 = jnp.sum(Mt0[:], axis=1, keepdims=True) - bcol_ref[:]
    probs1 = jnp.sum(Mt1[:], axis=1, keepdims=True) + bcol_ref[:]
    mx = jnp.maximum(probs0, probs1)
    e0 = jnp.exp(probs0 - mx)
    e1 = jnp.exp(probs1 - mx)
    s = e0 + e1
    out_ref[:, 0:1] = e0 / s
    out_ref[:, 1:2] = e1 / s


def kernel(J, b, W1, b1, W2, b2, W3, b3):
    J = J.astype(jnp.float32)
    b = b.astype(jnp.float32)
    bcol = b.reshape(N, 1)
    Jd = J.reshape(NB, BS, NB, BS)[jnp.arange(NB), :, jnp.arange(NB), :]
    Jd = Jd.reshape(N, BS)
    W1t = W1.T.astype(jnp.float32)
    b1c = b1.reshape(HID, 1).astype(jnp.float32)
    W2t = W2.T.astype(jnp.float32)
    b2c = b2.reshape(HID, 1).astype(jnp.float32)
    W3r = W3.T.astype(jnp.float32)
    b3s = b3.reshape(1).astype(jnp.float32)

    vmem = pl.BlockSpec(memory_space=pltpu.VMEM)
    smem = pl.BlockSpec(memory_space=pltpu.SMEM)
    return pl.pallas_call(
        _ggnn_kernel,
        out_shape=jax.ShapeDtypeStruct((N, 2), jnp.float32),
        in_specs=[vmem, smem, smem, vmem, vmem, vmem, vmem, vmem, vmem,
                  smem],
        out_specs=vmem,
        scratch_shapes=[pltpu.VMEM((N, N), jnp.float32)] * 6
        + [pltpu.VMEM((2 * BS, N), jnp.float32)] * 3
        + [pltpu.SMEM((2 * BS, N), jnp.float32)] * 3
        + [pltpu.SemaphoreType.DMA],
    )(J, b, Jd, bcol, W1t, b1c, W2t, b2c, W3r, b3s)
